# Initial kernel scaffold; baseline (speedup 1.0000x reference)
import jax
import jax.numpy as jnp
from jax.experimental import pallas as pl


def kernel(x, adj, W_heads, a_heads, W_out, a_out, gc_weight, gc_bias, enc_W, enc_b):
    raise NotImplementedError("write your pallas kernel here")


# speedup vs baseline: 2990.2118x; 2990.2118x over previous
import jax
import jax.numpy as jnp
from jax.experimental import pallas as pl

ALPHA = 0.2
BLK = 256


def _elu(v):
    return jnp.where(v > 0.0, v, jnp.exp(v) - 1.0)


def _prep_body(x_ref, wcat_ref, a1_ref, a2_ref,
               h_ref, f_ref, ef_ref, efa_ref, g_ref, eg_ref, ega_ref):
    h = jnp.dot(x_ref[...], wcat_ref[...], preferred_element_type=jnp.float32)
    h_ref[...] = h
    f = jnp.dot(h, a1_ref[...], preferred_element_type=jnp.float32)
    g = jnp.dot(h, a2_ref[...], preferred_element_type=jnp.float32)
    f_ref[...] = f
    g_ref[...] = g
    ef_ref[...] = jnp.exp(-f)
    efa_ref[...] = jnp.exp(-ALPHA * f)
    eg_ref[...] = jnp.exp(-g)
    ega_ref[...] = jnp.exp(-ALPHA * g)


def _att1_body(adj_ref, h_ref, f_ref, ef_ref, efa_ref,
               gt_ref, egt_ref, egat_ref, h1_ref):
    adj = adj_ref[...]
    nheads = f_ref.shape[1]
    nhid = h_ref.shape[1] // nheads
    for t in range(nheads):
        s_pos = (f_ref[:, t:t + 1] + gt_ref[t:t + 1, :]) > 0.0
        e = jnp.where(s_pos,
                      ef_ref[:, t:t + 1] * egt_ref[t:t + 1, :],
                      efa_ref[:, t:t + 1] * egat_ref[t:t + 1, :])
        e = adj * e
        rs = jnp.sum(e, axis=1, keepdims=True)
        num = jnp.dot(e, h_ref[:, t * nhid:(t + 1) * nhid],
                      preferred_element_type=jnp.float32)
        h1_ref[:, t * nhid:(t + 1) * nhid] = _elu(num / rs)


def _mid_body(h1_ref, gcw_ref, encw_ref, encb_ref, hg_ref, y_ref):
    h1 = h1_ref[...]
    hg_ref[...] = jnp.dot(h1, gcw_ref[...], preferred_element_type=jnp.float32)
    y_ref[...] = jnp.dot(h1, encw_ref[...],
                         preferred_element_type=jnp.float32) + encb_ref[...]


def _gcn_body(adj_ref, hg_ref, gcb_ref, wout_ref, encw_ref, encb_ref,
              ao1_ref, ao2_ref,
              z_ref, ho_ref, fo_ref, efo_ref, efao_ref, go_ref, ego_ref, egao_ref):
    h2 = jnp.dot(adj_ref[...], hg_ref[...],
                 preferred_element_type=jnp.float32) + gcb_ref[...]
    z_ref[...] = jnp.dot(h2, encw_ref[...],
                         preferred_element_type=jnp.float32) + encb_ref[...]
    ho = jnp.dot(h2, wout_ref[...], preferred_element_type=jnp.float32)
    ho_ref[...] = ho
    fo = jnp.dot(ho, ao1_ref[...], preferred_element_type=jnp.float32)
    go = jnp.dot(ho, ao2_ref[...], preferred_element_type=jnp.float32)
    fo_ref[...] = fo
    go_ref[...] = go
    efo_ref[...] = jnp.exp(-fo)
    efao_ref[...] = jnp.exp(-ALPHA * fo)
    ego_ref[...] = jnp.exp(-go)
    egao_ref[...] = jnp.exp(-ALPHA * go)


def _att2_body(adj_ref, ho_ref, fo_ref, efo_ref, efao_ref,
               got_ref, egot_ref, egaot_ref, out_ref):
    adj = adj_ref[...]
    s_pos = (fo_ref[...] + got_ref[...]) > 0.0
    e = jnp.where(s_pos,
                  efo_ref[...] * egot_ref[...],
                  efao_ref[...] * egaot_ref[...])
    e = adj * e
    rs = jnp.sum(e, axis=1, keepdims=True)
    num = jnp.dot(e, ho_ref[...], preferred_element_type=jnp.float32)
    xo = _elu(num / rs)
    m = jnp.max(xo, axis=1, keepdims=True)
    lse = m + jnp.log(jnp.sum(jnp.exp(xo - m), axis=1, keepdims=True))
    out_ref[...] = xo - lse


def kernel(x, adj, W_heads, a_heads, W_out, a_out, gc_weight, gc_bias, enc_W, enc_b):
    n, nfeat = x.shape
    nheads, _, nhid = W_heads.shape
    nh = nheads * nhid
    nclass = W_out.shape[1]
    nstruc = enc_W.shape[1]
    f32 = jnp.float32

    wcat = jnp.transpose(W_heads, (1, 0, 2)).reshape(nfeat, nh)
    a1 = a_heads[:, 0, :nhid]
    a2 = a_heads[:, 0, nhid:]
    eye = jnp.eye(nheads, dtype=f32)
    A1 = (a1[:, :, None] * eye[:, None, :]).reshape(nh, nheads)
    A2 = (a2[:, :, None] * eye[:, None, :]).reshape(nh, nheads)
    ao1 = a_out[0, :nclass].reshape(nclass, 1)
    ao2 = a_out[0, nclass:].reshape(nclass, 1)
    gcb = gc_bias.reshape(1, nh)
    encb = enc_b.reshape(1, nstruc)

    nblk = n // BLK
    full = lambda r, c: pl.BlockSpec((r, c), lambda i: (0, 0))
    rows = lambda c: pl.BlockSpec((BLK, c), lambda i: (i, 0))
    out_f32 = lambda r, c: jax.ShapeDtypeStruct((r, c), f32)

    h_all, f, ef, efa, g, eg, ega = pl.pallas_call(
        _prep_body,
        out_shape=[out_f32(n, nh)] + [out_f32(n, nheads)] * 6,
    )(x, wcat, A1, A2)
    gt, egt, egat = g.T, eg.T, ega.T

    h1 = pl.pallas_call(
        _att1_body,
        grid=(nblk,),
        in_specs=[rows(n), full(n, nh), rows(nheads), rows(nheads),
                  rows(nheads), full(nheads, n), full(nheads, n),
                  full(nheads, n)],
        out_specs=rows(nh),
        out_shape=out_f32(n, nh),
    )(adj, h_all, f, ef, efa, gt, egt, egat)

    hg, y = pl.pallas_call(
        _mid_body,
        out_shape=[out_f32(n, nh), out_f32(n, nstruc)],
    )(h1, gc_weight, enc_W, encb)

    z, ho, fo, efo, efao, go, ego, egao = pl.pallas_call(
        _gcn_body,
        grid=(nblk,),
        in_specs=[rows(n), full(n, nh), full(1, nh), full(nh, nclass),
                  full(nh, nstruc), full(1, nstruc), full(nclass, 1),
                  full(nclass, 1)],
        out_specs=[rows(nstruc), rows(nclass)] + [rows(1)] * 6,
        out_shape=[out_f32(n, nstruc), out_f32(n, nclass)] + [out_f32(n, 1)] * 6,
    )(adj, hg, gcb, W_out, enc_W, encb, ao1, ao2)
    got, egot, egaot = go.T, ego.T, egao.T

    xo = pl.pallas_call(
        _att2_body,
        grid=(nblk,),
        in_specs=[rows(n), full(n, nclass), rows(1), rows(1), rows(1),
                  full(1, n), full(1, n), full(1, n)],
        out_specs=rows(nclass),
        out_shape=out_f32(n, nclass),
    )(adj, ho, fo, efo, efao, got, egot, egaot)

    return (xo, y, z)


# speedup vs baseline: 3525.6305x; 1.1791x over previous
import jax
import jax.numpy as jnp
from jax.experimental import pallas as pl

ALPHA = 0.2
BLK = 256


def _elu(v):
    return jnp.where(v > 0.0, v, jnp.exp(v) - 1.0)


def _prep_body(x_ref, wcat_ref, a1_ref, a2_ref,
               f_ref, ef_ref, efa_ref, g_ref, uv_ref, v_ref):
    h = jnp.dot(x_ref[...], wcat_ref[...], preferred_element_type=jnp.float32)
    f = jnp.dot(h, a1_ref[...], preferred_element_type=jnp.float32)
    g = jnp.dot(h, a2_ref[...], preferred_element_type=jnp.float32)
    f_ref[...] = f
    g_ref[...] = g
    ef_ref[...] = jnp.exp(-f)
    efa_ref[...] = jnp.exp(-ALPHA * f)
    eg = jnp.exp(-g)
    ega = jnp.exp(-ALPHA * g)
    nheads = f.shape[1]
    nhid = h.shape[1] // nheads
    ones = jnp.ones_like(g[:, :1])
    uvs, vs = [], []
    for t in range(nheads):
        ht = h[:, t * nhid:(t + 1) * nhid]
        ut = jnp.concatenate([eg[:, t:t + 1] * ht, eg[:, t:t + 1]], axis=1)
        vt = jnp.concatenate([ega[:, t:t + 1] * ht, ega[:, t:t + 1]], axis=1)
        uvs += [ut, vt]
        vs.append(vt)
    uv_ref[...] = jnp.concatenate(uvs, axis=1)
    v_ref[...] = jnp.concatenate(vs, axis=1)
    del ones


def _att1_body(adj_ref, f_ref, ef_ref, efa_ref, gt_ref, uv_ref, v_ref, h1_ref):
    adj = adj_ref[...]
    nheads = f_ref.shape[1]
    w = uv_ref.shape[1] // (2 * nheads)
    nhid = w - 1
    av = jnp.dot(adj, v_ref[...], preferred_element_type=jnp.float32)
    for t in range(nheads):
        s_pos = (f_ref[:, t:t + 1] + gt_ref[t:t + 1, :]) > 0.0
        m = jnp.where(s_pos, adj, 0.0)
        nm = jnp.dot(m, uv_ref[:, 2 * w * t:2 * w * (t + 1)],
                     preferred_element_type=jnp.float32)
        acc = (ef_ref[:, t:t + 1] * nm[:, :w]
               + efa_ref[:, t:t + 1] * (av[:, w * t:w * (t + 1)] - nm[:, w:]))
        h1_ref[:, t * nhid:(t + 1) * nhid] = _elu(acc[:, :nhid] / acc[:, nhid:])


def _mid_body(h1_ref, gcw_ref, encw_ref, encb_ref, hg_ref, y_ref):
    h1 = h1_ref[...]
    hg_ref[...] = jnp.dot(h1, gcw_ref[...], preferred_element_type=jnp.float32)
    y_ref[...] = jnp.dot(h1, encw_ref[...],
                         preferred_element_type=jnp.float32) + encb_ref[...]


def _gcn_body(adj_ref, hg_ref, gcb_ref, wout_ref, encw_ref, encb_ref,
              ao1_ref, ao2_ref,
              z_ref, ho_ref, fo_ref, efo_ref, efao_ref, go_ref, uvo_ref, vo_ref):
    h2 = jnp.dot(adj_ref[...], hg_ref[...],
                 preferred_element_type=jnp.float32) + gcb_ref[...]
    z_ref[...] = jnp.dot(h2, encw_ref[...],
                         preferred_element_type=jnp.float32) + encb_ref[...]
    ho = jnp.dot(h2, wout_ref[...], preferred_element_type=jnp.float32)
    ho_ref[...] = ho
    fo = jnp.dot(ho, ao1_ref[...], preferred_element_type=jnp.float32)
    go = jnp.dot(ho, ao2_ref[...], preferred_element_type=jnp.float32)
    fo_ref[...] = fo
    go_ref[...] = go
    efo_ref[...] = jnp.exp(-fo)
    efao_ref[...] = jnp.exp(-ALPHA * fo)
    ego = jnp.exp(-go)
    egao = jnp.exp(-ALPHA * go)
    uo = jnp.concatenate([ego * ho, ego], axis=1)
    vo = jnp.concatenate([egao * ho, egao], axis=1)
    uvo_ref[...] = jnp.concatenate([uo, vo], axis=1)
    vo_ref[...] = vo


def _att2_body(adj_ref, fo_ref, efo_ref, efao_ref, got_ref, uvo_ref, vo_ref,
               out_ref):
    adj = adj_ref[...]
    w = vo_ref.shape[1]
    nclass = w - 1
    av = jnp.dot(adj, vo_ref[...], preferred_element_type=jnp.float32)
    s_pos = (fo_ref[...] + got_ref[...]) > 0.0
    m = jnp.where(s_pos, adj, 0.0)
    nm = jnp.dot(m, uvo_ref[...], preferred_element_type=jnp.float32)
    acc = efo_ref[...] * nm[:, :w] + efao_ref[...] * (av - nm[:, w:])
    xo = _elu(acc[:, :nclass] / acc[:, nclass:])
    mx = jnp.max(xo, axis=1, keepdims=True)
    lse = mx + jnp.log(jnp.sum(jnp.exp(xo - mx), axis=1, keepdims=True))
    out_ref[...] = xo - lse


def kernel(x, adj, W_heads, a_heads, W_out, a_out, gc_weight, gc_bias, enc_W, enc_b):
    n, nfeat = x.shape
    nheads, _, nhid = W_heads.shape
    nh = nheads * nhid
    nclass = W_out.shape[1]
    nstruc = enc_W.shape[1]
    w = nhid + 1
    wo = nclass + 1
    f32 = jnp.float32

    wcat = jnp.transpose(W_heads, (1, 0, 2)).reshape(nfeat, nh)
    a1 = a_heads[:, 0, :nhid]
    a2 = a_heads[:, 0, nhid:]
    eye = jnp.eye(nheads, dtype=f32)
    A1 = (a1[:, :, None] * eye[:, None, :]).reshape(nh, nheads)
    A2 = (a2[:, :, None] * eye[:, None, :]).reshape(nh, nheads)
    ao1 = a_out[0, :nclass].reshape(nclass, 1)
    ao2 = a_out[0, nclass:].reshape(nclass, 1)
    gcb = gc_bias.reshape(1, nh)
    encb = enc_b.reshape(1, nstruc)

    nblk = n // BLK
    full = lambda r, c: pl.BlockSpec((r, c), lambda i: (0, 0))
    rows = lambda c: pl.BlockSpec((BLK, c), lambda i: (i, 0))
    out_f32 = lambda r, c: jax.ShapeDtypeStruct((r, c), f32)

    f, ef, efa, g, uv, v = pl.pallas_call(
        _prep_body,
        out_shape=[out_f32(n, nheads)] * 4
        + [out_f32(n, 2 * w * nheads), out_f32(n, w * nheads)],
    )(x, wcat, A1, A2)
    gt = g.T

    h1 = pl.pallas_call(
        _att1_body,
        grid=(nblk,),
        in_specs=[rows(n), rows(nheads), rows(nheads), rows(nheads),
                  full(nheads, n), full(n, 2 * w * nheads), full(n, w * nheads)],
        out_specs=rows(nh),
        out_shape=out_f32(n, nh),
    )(adj, f, ef, efa, gt, uv, v)

    hg, y = pl.pallas_call(
        _mid_body,
        out_shape=[out_f32(n, nh), out_f32(n, nstruc)],
    )(h1, gc_weight, enc_W, encb)

    z, ho, fo, efo, efao, go, uvo, vo = pl.pallas_call(
        _gcn_body,
        grid=(nblk,),
        in_specs=[rows(n), full(n, nh), full(1, nh), full(nh, nclass),
                  full(nh, nstruc), full(1, nstruc), full(nclass, 1),
                  full(nclass, 1)],
        out_specs=[rows(nstruc), rows(nclass)] + [rows(1)] * 4
        + [rows(2 * wo), rows(wo)],
        out_shape=[out_f32(n, nstruc), out_f32(n, nclass)]
        + [out_f32(n, 1)] * 4 + [out_f32(n, 2 * wo), out_f32(n, wo)],
    )(adj, hg, gcb, W_out, enc_W, encb, ao1, ao2)
    got = go.T

    xo = pl.pallas_call(
        _att2_body,
        grid=(nblk,),
        in_specs=[rows(n), rows(1), rows(1), rows(1),
                  full(1, n), full(n, 2 * wo), full(n, wo)],
        out_specs=rows(nclass),
        out_shape=out_f32(n, nclass),
    )(adj, fo, efo, efao, got, uvo, vo)

    return (xo, y, z)


# speedup vs baseline: 3819.1538x; 1.0833x over previous
import jax
import jax.numpy as jnp
from jax.experimental import pallas as pl

ALPHA = 0.2
BLK = 256


def _elu(v):
    return jnp.where(v > 0.0, v, jnp.exp(v) - 1.0)


def _prep_body(x_ref, wcat_ref, a1_ref, a2_ref, r8_ref,
               f_ref, ef_ref, efa_ref, g_ref, u_ref, v_ref):
    h = jnp.dot(x_ref[...], wcat_ref[...], preferred_element_type=jnp.float32)
    f = jnp.dot(h, a1_ref[...], preferred_element_type=jnp.float32)
    g = jnp.dot(h, a2_ref[...], preferred_element_type=jnp.float32)
    f_ref[...] = f
    g_ref[...] = g
    ef_ref[...] = jnp.exp(-f)
    efa_ref[...] = jnp.exp(-ALPHA * f)
    eg = jnp.exp(-g)
    ega = jnp.exp(-ALPHA * g)
    nh = h.shape[1]
    u_ref[:, :nh] = jnp.dot(eg, r8_ref[...],
                            preferred_element_type=jnp.float32) * h
    u_ref[:, nh:] = eg
    v_ref[:, :nh] = jnp.dot(ega, r8_ref[...],
                            preferred_element_type=jnp.float32) * h
    v_ref[:, nh:] = ega


def _att1_body(adj_ref, f_ref, ef_ref, efa_ref, gt_ref, u_ref, v_ref, h1_ref):
    adj = adj_ref[...]
    nheads = f_ref.shape[1]
    nh = u_ref.shape[1] - nheads
    nhid = nh // nheads
    av = jnp.dot(adj, v_ref[...], preferred_element_type=jnp.float32)
    for t in range(nheads):
        s_pos = (f_ref[:, t:t + 1] + gt_ref[t:t + 1, :]) > 0.0
        m = jnp.where(s_pos, adj, 0.0)
        nu = jnp.dot(m, u_ref[...], preferred_element_type=jnp.float32)
        nv = jnp.dot(m, v_ref[...], preferred_element_type=jnp.float32)
        ef = ef_ref[:, t:t + 1]
        efa = efa_ref[:, t:t + 1]
        lo, hi = t * nhid, (t + 1) * nhid
        num = ef * nu[:, lo:hi] + efa * (av[:, lo:hi] - nv[:, lo:hi])
        den = (ef * nu[:, nh + t:nh + t + 1]
               + efa * (av[:, nh + t:nh + t + 1] - nv[:, nh + t:nh + t + 1]))
        h1_ref[:, lo:hi] = _elu(num / den)


def _mid_body(h1_ref, gcw_ref, encw_ref, encb_ref, hg_ref, y_ref):
    h1 = h1_ref[...]
    hg_ref[...] = jnp.dot(h1, gcw_ref[...], preferred_element_type=jnp.float32)
    y_ref[...] = jnp.dot(h1, encw_ref[...],
                         preferred_element_type=jnp.float32) + encb_ref[...]


def _gcn_body(adj_ref, hg_ref, gcb_ref, wout_ref, encw_ref, encb_ref,
              ao1_ref, ao2_ref,
              z_ref, ho_ref, fo_ref, efo_ref, efao_ref, go_ref, uvo_ref, vo_ref):
    h2 = jnp.dot(adj_ref[...], hg_ref[...],
                 preferred_element_type=jnp.float32) + gcb_ref[...]
    z_ref[...] = jnp.dot(h2, encw_ref[...],
                         preferred_element_type=jnp.float32) + encb_ref[...]
    ho = jnp.dot(h2, wout_ref[...], preferred_element_type=jnp.float32)
    ho_ref[...] = ho
    fo = jnp.dot(ho, ao1_ref[...], preferred_element_type=jnp.float32)
    go = jnp.dot(ho, ao2_ref[...], preferred_element_type=jnp.float32)
    fo_ref[...] = fo
    go_ref[...] = go
    efo_ref[...] = jnp.exp(-fo)
    efao_ref[...] = jnp.exp(-ALPHA * fo)
    ego = jnp.exp(-go)
    egao = jnp.exp(-ALPHA * go)
    nclass = ho.shape[1]
    uvo_ref[:, :nclass] = ego * ho
    uvo_ref[:, nclass:nclass + 1] = ego
    uvo_ref[:, nclass + 1:2 * nclass + 1] = egao * ho
    uvo_ref[:, 2 * nclass + 1:] = egao
    vo_ref[:, :nclass] = egao * ho
    vo_ref[:, nclass:] = egao


def _att2_body(adj_ref, fo_ref, efo_ref, efao_ref, got_ref, uvo_ref, vo_ref,
               out_ref):
    adj = adj_ref[...]
    w = vo_ref.shape[1]
    nclass = w - 1
    av = jnp.dot(adj, vo_ref[...], preferred_element_type=jnp.float32)
    s_pos = (fo_ref[...] + got_ref[...]) > 0.0
    m = jnp.where(s_pos, adj, 0.0)
    nm = jnp.dot(m, uvo_ref[...], preferred_element_type=jnp.float32)
    acc = efo_ref[...] * nm[:, :w] + efao_ref[...] * (av - nm[:, w:])
    xo = _elu(acc[:, :nclass] / acc[:, nclass:])
    mx = jnp.max(xo, axis=1, keepdims=True)
    lse = mx + jnp.log(jnp.sum(jnp.exp(xo - mx), axis=1, keepdims=True))
    out_ref[...] = xo - lse


def kernel(x, adj, W_heads, a_heads, W_out, a_out, gc_weight, gc_bias, enc_W, enc_b):
    n, nfeat = x.shape
    nheads, _, nhid = W_heads.shape
    nh = nheads * nhid
    nclass = W_out.shape[1]
    nstruc = enc_W.shape[1]
    w = nhid + 1
    wo = nclass + 1
    f32 = jnp.float32

    wcat = jnp.transpose(W_heads, (1, 0, 2)).reshape(nfeat, nh)
    a1 = a_heads[:, 0, :nhid]
    a2 = a_heads[:, 0, nhid:]
    eye = jnp.eye(nheads, dtype=f32)
    A1 = (a1[:, :, None] * eye[:, None, :]).reshape(nh, nheads)
    A2 = (a2[:, :, None] * eye[:, None, :]).reshape(nh, nheads)
    ao1 = a_out[0, :nclass].reshape(nclass, 1)
    ao2 = a_out[0, nclass:].reshape(nclass, 1)
    gcb = gc_bias.reshape(1, nh)
    encb = enc_b.reshape(1, nstruc)

    nblk = n // BLK
    full = lambda r, c: pl.BlockSpec((r, c), lambda i: (0, 0))
    rows = lambda c: pl.BlockSpec((BLK, c), lambda i: (i, 0))
    out_f32 = lambda r, c: jax.ShapeDtypeStruct((r, c), f32)

    R8 = jnp.repeat(eye, nhid, axis=1)

    f, ef, efa, g, u, v = pl.pallas_call(
        _prep_body,
        out_shape=[out_f32(n, nheads)] * 4
        + [out_f32(n, nh + nheads), out_f32(n, nh + nheads)],
    )(x, wcat, A1, A2, R8)
    gt = g.T

    h1 = pl.pallas_call(
        _att1_body,
        grid=(nblk,),
        in_specs=[rows(n), rows(nheads), rows(nheads), rows(nheads),
                  full(nheads, n), full(n, nh + nheads), full(n, nh + nheads)],
        out_specs=rows(nh),
        out_shape=out_f32(n, nh),
    )(adj, f, ef, efa, gt, u, v)

    hg, y = pl.pallas_call(
        _mid_body,
        out_shape=[out_f32(n, nh), out_f32(n, nstruc)],
    )(h1, gc_weight, enc_W, encb)

    z, ho, fo, efo, efao, go, uvo, vo = pl.pallas_call(
        _gcn_body,
        grid=(nblk,),
        in_specs=[rows(n), full(n, nh), full(1, nh), full(nh, nclass),
                  full(nh, nstruc), full(1, nstruc), full(nclass, 1),
                  full(nclass, 1)],
        out_specs=[rows(nstruc), rows(nclass)] + [rows(1)] * 4
        + [rows(2 * wo), rows(wo)],
        out_shape=[out_f32(n, nstruc), out_f32(n, nclass)]
        + [out_f32(n, 1)] * 4 + [out_f32(n, 2 * wo), out_f32(n, wo)],
    )(adj, hg, gcb, W_out, enc_W, encb, ao1, ao2)
    got = go.T

    xo = pl.pallas_call(
        _att2_body,
        grid=(nblk,),
        in_specs=[rows(n), rows(1), rows(1), rows(1),
                  full(1, n), full(n, 2 * wo), full(n, wo)],
        out_specs=rows(nclass),
        out_shape=out_f32(n, nclass),
    )(adj, fo, efo, efao, got, uvo, vo)

    return (xo, y, z)


# speedup vs baseline: 3969.9277x; 1.0395x over previous
import jax
import jax.numpy as jnp
from jax.experimental import pallas as pl

ALPHA = 0.2
BLK = 256
BF16 = jnp.bfloat16


def _elu(v):
    return jnp.where(v > 0.0, v, jnp.exp(v) - 1.0)


def _prep_body(x_ref, wcat_ref, a1_ref, a2_ref, r8_ref,
               f_ref, ef_ref, efa_ref, g_ref, u_ref, v_ref):
    h = jnp.dot(x_ref[...], wcat_ref[...], preferred_element_type=jnp.float32)
    f = jnp.dot(h, a1_ref[...], preferred_element_type=jnp.float32)
    g = jnp.dot(h, a2_ref[...], preferred_element_type=jnp.float32)
    f_ref[...] = f
    g_ref[...] = g
    ef_ref[...] = jnp.exp(-f)
    efa_ref[...] = jnp.exp(-ALPHA * f)
    eg = jnp.exp(-g)
    ega = jnp.exp(-ALPHA * g)
    nh = h.shape[1]
    u_ref[:, :nh] = (jnp.dot(eg, r8_ref[...],
                             preferred_element_type=jnp.float32) * h).astype(BF16)
    u_ref[:, nh:] = eg.astype(BF16)
    v_ref[:, :nh] = (jnp.dot(ega, r8_ref[...],
                             preferred_element_type=jnp.float32) * h).astype(BF16)
    v_ref[:, nh:] = ega.astype(BF16)


def _att1_body(adj_ref, f_ref, ef_ref, efa_ref, gt_ref, u_ref, v_ref,
               h1_ref, adj16_ref):
    adjh = adj_ref[...].astype(BF16)
    adj16_ref[...] = adjh
    nheads = f_ref.shape[1]
    nh = u_ref.shape[1] - nheads
    nhid = nh // nheads
    u16 = u_ref[...]
    v16 = v_ref[...]
    av = jnp.dot(adjh, v16, preferred_element_type=jnp.float32)
    for t in range(nheads):
        s_pos = (f_ref[:, t:t + 1] + gt_ref[t:t + 1, :]) > 0.0
        m = jnp.where(s_pos, adjh, jnp.zeros_like(adjh))
        nu = jnp.dot(m, u16, preferred_element_type=jnp.float32)
        nv = jnp.dot(m, v16, preferred_element_type=jnp.float32)
        ef = ef_ref[:, t:t + 1]
        efa = efa_ref[:, t:t + 1]
        lo, hi = t * nhid, (t + 1) * nhid
        num = ef * nu[:, lo:hi] + efa * (av[:, lo:hi] - nv[:, lo:hi])
        den = (ef * nu[:, nh + t:nh + t + 1]
               + efa * (av[:, nh + t:nh + t + 1] - nv[:, nh + t:nh + t + 1]))
        h1_ref[:, lo:hi] = _elu(num / den)


def _mid_body(h1_ref, gcw_ref, encw_ref, encb_ref, hg_ref, y_ref):
    h1 = h1_ref[...]
    hg_ref[...] = jnp.dot(h1, gcw_ref[...],
                          preferred_element_type=jnp.float32).astype(BF16)
    y_ref[...] = jnp.dot(h1, encw_ref[...],
                         preferred_element_type=jnp.float32) + encb_ref[...]


def _gcn_body(adj16_ref, hg_ref, gcb_ref, wout_ref, encw_ref, encb_ref,
              ao1_ref, ao2_ref,
              z_ref, fo_ref, efo_ref, efao_ref, go_ref, uvo_ref, vo_ref):
    h2 = jnp.dot(adj16_ref[...], hg_ref[...],
                 preferred_element_type=jnp.float32) + gcb_ref[...]
    z_ref[...] = jnp.dot(h2, encw_ref[...],
                         preferred_element_type=jnp.float32) + encb_ref[...]
    ho = jnp.dot(h2, wout_ref[...], preferred_element_type=jnp.float32)
    fo = jnp.dot(ho, ao1_ref[...], preferred_element_type=jnp.float32)
    go = jnp.dot(ho, ao2_ref[...], preferred_element_type=jnp.float32)
    fo_ref[...] = fo
    go_ref[...] = go
    efo_ref[...] = jnp.exp(-fo)
    efao_ref[...] = jnp.exp(-ALPHA * fo)
    ego = jnp.exp(-go)
    egao = jnp.exp(-ALPHA * go)
    nclass = ho.shape[1]
    uvo_ref[:, :nclass] = (ego * ho).astype(BF16)
    uvo_ref[:, nclass:nclass + 1] = ego.astype(BF16)
    uvo_ref[:, nclass + 1:2 * nclass + 1] = (egao * ho).astype(BF16)
    uvo_ref[:, 2 * nclass + 1:] = egao.astype(BF16)
    vo_ref[:, :nclass] = (egao * ho).astype(BF16)
    vo_ref[:, nclass:] = egao.astype(BF16)


def _att2_body(adj16_ref, fo_ref, efo_ref, efao_ref, got_ref, uvo_ref, vo_ref,
               out_ref):
    adjh = adj16_ref[...]
    w = vo_ref.shape[1]
    nclass = w - 1
    av = jnp.dot(adjh, vo_ref[...], preferred_element_type=jnp.float32)
    s_pos = (fo_ref[...] + got_ref[...]) > 0.0
    m = jnp.where(s_pos, adjh, jnp.zeros_like(adjh))
    nm = jnp.dot(m, uvo_ref[...], preferred_element_type=jnp.float32)
    acc = efo_ref[...] * nm[:, :w] + efao_ref[...] * (av - nm[:, w:])
    xo = _elu(acc[:, :nclass] / acc[:, nclass:])
    mx = jnp.max(xo, axis=1, keepdims=True)
    lse = mx + jnp.log(jnp.sum(jnp.exp(xo - mx), axis=1, keepdims=True))
    out_ref[...] = xo - lse


def kernel(x, adj, W_heads, a_heads, W_out, a_out, gc_weight, gc_bias, enc_W, enc_b):
    n, nfeat = x.shape
    nheads, _, nhid = W_heads.shape
    nh = nheads * nhid
    nclass = W_out.shape[1]
    nstruc = enc_W.shape[1]
    wo = nclass + 1
    f32 = jnp.float32

    wcat = jnp.transpose(W_heads, (1, 0, 2)).reshape(nfeat, nh)
    a1 = a_heads[:, 0, :nhid]
    a2 = a_heads[:, 0, nhid:]
    eye = jnp.eye(nheads, dtype=f32)
    A1 = (a1[:, :, None] * eye[:, None, :]).reshape(nh, nheads)
    A2 = (a2[:, :, None] * eye[:, None, :]).reshape(nh, nheads)
    ao1 = a_out[0, :nclass].reshape(nclass, 1)
    ao2 = a_out[0, nclass:].reshape(nclass, 1)
    gcb = gc_bias.reshape(1, nh)
    encb = enc_b.reshape(1, nstruc)
    R8 = jnp.repeat(eye, nhid, axis=1)

    nblk = n // BLK
    full = lambda r, c: pl.BlockSpec((r, c), lambda i: (0, 0))
    rows = lambda c: pl.BlockSpec((BLK, c), lambda i: (i, 0))
    out_f32 = lambda r, c: jax.ShapeDtypeStruct((r, c), f32)
    out_bf16 = lambda r, c: jax.ShapeDtypeStruct((r, c), BF16)

    f, ef, efa, g, u, v = pl.pallas_call(
        _prep_body,
        out_shape=[out_f32(n, nheads)] * 4
        + [out_bf16(n, nh + nheads), out_bf16(n, nh + nheads)],
    )(x, wcat, A1, A2, R8)
    gt = g.T

    h1, adj16 = pl.pallas_call(
        _att1_body,
        grid=(nblk,),
        in_specs=[rows(n), rows(nheads), rows(nheads), rows(nheads),
                  full(nheads, n), full(n, nh + nheads), full(n, nh + nheads)],
        out_specs=[rows(nh), rows(n)],
        out_shape=[out_f32(n, nh), out_bf16(n, n)],
    )(adj, f, ef, efa, gt, u, v)

    hg, y = pl.pallas_call(
        _mid_body,
        out_shape=[out_bf16(n, nh), out_f32(n, nstruc)],
    )(h1, gc_weight, enc_W, encb)

    z, fo, efo, efao, go, uvo, vo = pl.pallas_call(
        _gcn_body,
        grid=(nblk,),
        in_specs=[rows(n), full(n, nh), full(1, nh), full(nh, nclass),
                  full(nh, nstruc), full(1, nstruc), full(nclass, 1),
                  full(nclass, 1)],
        out_specs=[rows(nstruc)] + [rows(1)] * 4 + [rows(2 * wo), rows(wo)],
        out_shape=[out_f32(n, nstruc)] + [out_f32(n, 1)] * 4
        + [out_bf16(n, 2 * wo), out_bf16(n, wo)],
    )(adj16, hg, gcb, W_out, enc_W, encb, ao1, ao2)
    got = go.T

    xo = pl.pallas_call(
        _att2_body,
        grid=(nblk,),
        in_specs=[rows(n), rows(1), rows(1), rows(1),
                  full(1, n), full(n, 2 * wo), full(n, wo)],
        out_specs=rows(nclass),
        out_shape=out_f32(n, nclass),
    )(adj16, fo, efo, efao, got, uvo, vo)

    return (xo, y, z)


# speedup vs baseline: 4408.9152x; 1.1106x over previous
import jax
import jax.numpy as jnp
from jax.experimental import pallas as pl
from jax.experimental.pallas import tpu as pltpu

ALPHA = 0.2
BLK = 256
BF16 = jnp.bfloat16
F32 = jnp.float32


def _elu(v):
    return jnp.where(v > 0.0, v, jnp.exp(v) - 1.0)


def _att1_fused_body(adj_ref, x_ref, wcat_ref, a1_ref, a2_ref, r8_ref,
                     encw_ref, encb_ref,
                     h1_ref, adj16_ref, y_ref,
                     f_s, ef_s, efa_s, gt_s, u_s, v_s):
    i = pl.program_id(0)
    nheads = f_s.shape[1]
    nh = u_s.shape[1] - nheads
    nhid = nh // nheads

    @pl.when(i == 0)
    def _prologue():
        h = jnp.dot(x_ref[...], wcat_ref[...], preferred_element_type=F32)
        f = jnp.dot(h, a1_ref[...], preferred_element_type=F32)
        g = jnp.dot(h, a2_ref[...], preferred_element_type=F32)
        f_s[...] = f
        ef_s[...] = jnp.exp(-f)
        efa_s[...] = jnp.exp(-ALPHA * f)
        gt_s[...] = jnp.transpose(g)
        eg = jnp.exp(-g)
        ega = jnp.exp(-ALPHA * g)
        u_s[:, :nh] = (jnp.dot(eg, r8_ref[...],
                               preferred_element_type=F32) * h).astype(BF16)
        u_s[:, nh:] = eg.astype(BF16)
        v_s[:, :nh] = (jnp.dot(ega, r8_ref[...],
                               preferred_element_type=F32) * h).astype(BF16)
        v_s[:, nh:] = ega.astype(BF16)

    adjh = adj_ref[...].astype(BF16)
    adj16_ref[...] = adjh
    u16 = u_s[...]
    v16 = v_s[...]
    r = pl.ds(i * BLK, BLK)
    fr = f_s[r, :]
    efr = ef_s[r, :]
    efar = efa_s[r, :]
    av = jnp.dot(adjh, v16, preferred_element_type=F32)
    for t in range(nheads):
        s_pos = (fr[:, t:t + 1] + gt_s[t:t + 1, :]) > 0.0
        m = jnp.where(s_pos, adjh, jnp.zeros_like(adjh))
        nu = jnp.dot(m, u16, preferred_element_type=F32)
        nv = jnp.dot(m, v16, preferred_element_type=F32)
        ef = efr[:, t:t + 1]
        efa = efar[:, t:t + 1]
        lo, hi = t * nhid, (t + 1) * nhid
        num = ef * nu[:, lo:hi] + efa * (av[:, lo:hi] - nv[:, lo:hi])
        den = (ef * nu[:, nh + t:nh + t + 1]
               + efa * (av[:, nh + t:nh + t + 1] - nv[:, nh + t:nh + t + 1]))
        h1_ref[:, lo:hi] = _elu(num / den)
    y_ref[...] = jnp.dot(h1_ref[...], encw_ref[...],
                         preferred_element_type=F32) + encb_ref[...]


def _att2_fused_body(adj16_ref, h1_ref, gcw_ref, gcb_ref, wout_ref,
                     encw_ref, encb_ref, ao1_ref, ao2_ref,
                     z_ref, xo_ref,
                     hg_s, h2_s, fo_s, efo_s, efao_s, go_s, got_s, uvo_s, vo_s):
    p = pl.program_id(0)
    i = pl.program_id(1)
    nclass = wout_ref.shape[1]
    w = nclass + 1
    r = pl.ds(i * BLK, BLK)

    @pl.when((p == 0) & (i == 0))
    def _prologue():
        hg_s[...] = jnp.dot(h1_ref[...], gcw_ref[...],
                            preferred_element_type=F32).astype(BF16)

    @pl.when(p == 0)
    def _phase0():
        h2 = jnp.dot(adj16_ref[...], hg_s[...],
                     preferred_element_type=F32) + gcb_ref[...]
        h2_s[r, :] = h2
        ho = jnp.dot(h2, wout_ref[...], preferred_element_type=F32)
        fo = jnp.dot(ho, ao1_ref[...], preferred_element_type=F32)
        go = jnp.dot(ho, ao2_ref[...], preferred_element_type=F32)
        fo_s[r, :] = fo
        go_s[r, :] = go
        efo_s[r, :] = jnp.exp(-fo)
        efao_s[r, :] = jnp.exp(-ALPHA * fo)
        ego = jnp.exp(-go)
        egao = jnp.exp(-ALPHA * go)
        uvo_s[r, :nclass] = (ego * ho).astype(BF16)
        uvo_s[r, nclass:w] = ego.astype(BF16)
        uvo_s[r, w:w + nclass] = (egao * ho).astype(BF16)
        uvo_s[r, w + nclass:] = egao.astype(BF16)
        vo_s[r, :nclass] = (egao * ho).astype(BF16)
        vo_s[r, nclass:] = egao.astype(BF16)

    @pl.when((p == 1) & (i == 0))
    def _transpose_go():
        got_s[...] = jnp.transpose(go_s[...])

    @pl.when(p == 1)
    def _phase1():
        adjh = adj16_ref[...]
        av = jnp.dot(adjh, vo_s[...], preferred_element_type=F32)
        s_pos = (fo_s[r, :] + got_s[...]) > 0.0
        m = jnp.where(s_pos, adjh, jnp.zeros_like(adjh))
        nm = jnp.dot(m, uvo_s[...], preferred_element_type=F32)
        acc = (efo_s[r, :] * nm[:, :w]
               + efao_s[r, :] * (av - nm[:, w:]))
        xo = _elu(acc[:, :nclass] / acc[:, nclass:])
        mx = jnp.max(xo, axis=1, keepdims=True)
        lse = mx + jnp.log(jnp.sum(jnp.exp(xo - mx), axis=1, keepdims=True))
        xo_ref[0] = xo - lse
        z_ref[0] = jnp.dot(h2_s[r, :], encw_ref[...],
                           preferred_element_type=F32) + encb_ref[...]


def kernel(x, adj, W_heads, a_heads, W_out, a_out, gc_weight, gc_bias, enc_W, enc_b):
    n, nfeat = x.shape
    nheads, _, nhid = W_heads.shape
    nh = nheads * nhid
    nclass = W_out.shape[1]
    nstruc = enc_W.shape[1]
    wo = nclass + 1

    wcat = jnp.transpose(W_heads, (1, 0, 2)).reshape(nfeat, nh)
    a1 = a_heads[:, 0, :nhid]
    a2 = a_heads[:, 0, nhid:]
    eye = jnp.eye(nheads, dtype=F32)
    A1 = (a1[:, :, None] * eye[:, None, :]).reshape(nh, nheads)
    A2 = (a2[:, :, None] * eye[:, None, :]).reshape(nh, nheads)
    ao1 = a_out[0, :nclass].reshape(nclass, 1)
    ao2 = a_out[0, nclass:].reshape(nclass, 1)
    gcb = gc_bias.reshape(1, nh)
    encb = enc_b.reshape(1, nstruc)
    R8 = jnp.repeat(eye, nhid, axis=1)

    nblk = n // BLK
    out_f32 = lambda r, c: jax.ShapeDtypeStruct((r, c), F32)

    h1, adj16, y = pl.pallas_call(
        _att1_fused_body,
        grid=(nblk,),
        in_specs=[pl.BlockSpec((BLK, n), lambda i: (i, 0)),
                  pl.BlockSpec((n, nfeat), lambda i: (0, 0)),
                  pl.BlockSpec((nfeat, nh), lambda i: (0, 0)),
                  pl.BlockSpec((nh, nheads), lambda i: (0, 0)),
                  pl.BlockSpec((nh, nheads), lambda i: (0, 0)),
                  pl.BlockSpec((nheads, nh), lambda i: (0, 0)),
                  pl.BlockSpec((nh, nstruc), lambda i: (0, 0)),
                  pl.BlockSpec((1, nstruc), lambda i: (0, 0))],
        out_specs=[pl.BlockSpec((BLK, nh), lambda i: (i, 0)),
                   pl.BlockSpec((BLK, n), lambda i: (i, 0)),
                   pl.BlockSpec((BLK, nstruc), lambda i: (i, 0))],
        out_shape=[out_f32(n, nh), jax.ShapeDtypeStruct((n, n), BF16),
                   out_f32(n, nstruc)],
        scratch_shapes=[pltpu.VMEM((n, nheads), F32),
                        pltpu.VMEM((n, nheads), F32),
                        pltpu.VMEM((n, nheads), F32),
                        pltpu.VMEM((nheads, n), F32),
                        pltpu.VMEM((n, nh + nheads), BF16),
                        pltpu.VMEM((n, nh + nheads), BF16)],
    )(adj, x, wcat, A1, A2, R8, enc_W, encb)

    z2, xo2 = pl.pallas_call(
        _att2_fused_body,
        grid=(2, nblk),
        in_specs=[pl.BlockSpec((BLK, n), lambda p, i: (i, 0)),
                  pl.BlockSpec((n, nh), lambda p, i: (0, 0)),
                  pl.BlockSpec((nh, nh), lambda p, i: (0, 0)),
                  pl.BlockSpec((1, nh), lambda p, i: (0, 0)),
                  pl.BlockSpec((nh, nclass), lambda p, i: (0, 0)),
                  pl.BlockSpec((nh, nstruc), lambda p, i: (0, 0)),
                  pl.BlockSpec((1, nstruc), lambda p, i: (0, 0)),
                  pl.BlockSpec((nclass, 1), lambda p, i: (0, 0)),
                  pl.BlockSpec((nclass, 1), lambda p, i: (0, 0))],
        out_specs=[pl.BlockSpec((1, BLK, nstruc), lambda p, i: (p, i, 0)),
                   pl.BlockSpec((1, BLK, nclass), lambda p, i: (p, i, 0))],
        out_shape=[jax.ShapeDtypeStruct((2, n, nstruc), F32),
                   jax.ShapeDtypeStruct((2, n, nclass), F32)],
        scratch_shapes=[pltpu.VMEM((n, nh), BF16),
                        pltpu.VMEM((n, nh), F32),
                        pltpu.VMEM((n, 1), F32),
                        pltpu.VMEM((n, 1), F32),
                        pltpu.VMEM((n, 1), F32),
                        pltpu.VMEM((n, 1), F32),
                        pltpu.VMEM((1, n), F32),
                        pltpu.VMEM((n, 2 * wo), BF16),
                        pltpu.VMEM((n, wo), BF16)],
    )(adj16, h1, gc_weight, gcb, W_out, enc_W, encb, ao1, ao2)

    return (xo2[1], y, z2[1])


# speedup vs baseline: 4514.4555x; 1.0239x over previous
import jax
import jax.numpy as jnp
from jax.experimental import pallas as pl
from jax.experimental.pallas import tpu as pltpu

ALPHA = 0.2
BLK = 256
BF16 = jnp.bfloat16
F32 = jnp.float32


def _elu(v):
    return jnp.where(v > 0.0, v, jnp.exp(v) - 1.0)


def _gat_body(adj_ref, x_ref, wcat_ref, a1_ref, a2_ref, r8_ref,
              gcw_ref, gcb_ref, wout_ref, encw_ref, encb_ref, ao1_ref, ao2_ref,
              y_ref, z_ref, xo_ref,
              adj16_s, f_s, ef_s, efa_s, gt_s, u_s, v_s, h1_s, hg_s, h2_s,
              fo_s, efo_s, efao_s, go_s, got_s, uvo_s, vo_s):
    p = pl.program_id(0)
    i = pl.program_id(1)
    nheads = f_s.shape[1]
    nh = u_s.shape[1] - nheads
    nhid = nh // nheads
    nclass = wout_ref.shape[1]
    w = nclass + 1
    r = pl.ds(i * BLK, BLK)

    @pl.when((p == 0) & (i == 0))
    def _prologue_a():
        h = jnp.dot(x_ref[...], wcat_ref[...], preferred_element_type=F32)
        f = jnp.dot(h, a1_ref[...], preferred_element_type=F32)
        g = jnp.dot(h, a2_ref[...], preferred_element_type=F32)
        f_s[...] = f
        ef_s[...] = jnp.exp(-f)
        efa_s[...] = jnp.exp(-ALPHA * f)
        gt_s[...] = jnp.transpose(g)
        eg = jnp.exp(-g)
        ega = jnp.exp(-ALPHA * g)
        u_s[:, :nh] = (jnp.dot(eg, r8_ref[...],
                               preferred_element_type=F32) * h).astype(BF16)
        u_s[:, nh:] = eg.astype(BF16)
        v_s[:, :nh] = (jnp.dot(ega, r8_ref[...],
                               preferred_element_type=F32) * h).astype(BF16)
        v_s[:, nh:] = ega.astype(BF16)

    @pl.when(p == 0)
    def _phase0():
        adjh = adj_ref[...].astype(BF16)
        adj16_s[r, :] = adjh
        u16 = u_s[...]
        v16 = v_s[...]
        fr = f_s[r, :]
        efr = ef_s[r, :]
        efar = efa_s[r, :]
        av = jnp.dot(adjh, v16, preferred_element_type=F32)
        for t in range(nheads):
            s_pos = (fr[:, t:t + 1] + gt_s[t:t + 1, :]) > 0.0
            m = jnp.where(s_pos, adjh, jnp.zeros_like(adjh))
            nu = jnp.dot(m, u16, preferred_element_type=F32)
            nv = jnp.dot(m, v16, preferred_element_type=F32)
            ef = efr[:, t:t + 1]
            efa = efar[:, t:t + 1]
            lo, hi = t * nhid, (t + 1) * nhid
            num = ef * nu[:, lo:hi] + efa * (av[:, lo:hi] - nv[:, lo:hi])
            den = (ef * nu[:, nh + t:nh + t + 1]
                   + efa * (av[:, nh + t:nh + t + 1] - nv[:, nh + t:nh + t + 1]))
            h1_s[r, lo:hi] = _elu(num / den)
        y_ref[0] = jnp.dot(h1_s[r, :], encw_ref[...],
                           preferred_element_type=F32) + encb_ref[...]

    @pl.when((p == 1) & (i == 0))
    def _prologue_b():
        hg_s[...] = jnp.dot(h1_s[...], gcw_ref[...],
                            preferred_element_type=F32).astype(BF16)

    @pl.when(p == 1)
    def _phase1():
        h2 = jnp.dot(adj16_s[r, :], hg_s[...],
                     preferred_element_type=F32) + gcb_ref[...]
        h2_s[r, :] = h2
        z_ref[0] = jnp.dot(h2, encw_ref[...],
                           preferred_element_type=F32) + encb_ref[...]
        ho = jnp.dot(h2, wout_ref[...], preferred_element_type=F32)
        fo = jnp.dot(ho, ao1_ref[...], preferred_element_type=F32)
        go = jnp.dot(ho, ao2_ref[...], preferred_element_type=F32)
        fo_s[r, :] = fo
        go_s[r, :] = go
        efo_s[r, :] = jnp.exp(-fo)
        efao_s[r, :] = jnp.exp(-ALPHA * fo)
        ego = jnp.exp(-go)
        egao = jnp.exp(-ALPHA * go)
        uvo_s[r, :nclass] = (ego * ho).astype(BF16)
        uvo_s[r, nclass:w] = ego.astype(BF16)
        uvo_s[r, w:w + nclass] = (egao * ho).astype(BF16)
        uvo_s[r, w + nclass:] = egao.astype(BF16)
        vo_s[r, :nclass] = (egao * ho).astype(BF16)
        vo_s[r, nclass:] = egao.astype(BF16)

    @pl.when((p == 2) & (i == 0))
    def _transpose_go():
        got_s[...] = jnp.transpose(go_s[...])

    @pl.when(p == 2)
    def _phase2():
        adjh = adj16_s[r, :]
        av = jnp.dot(adjh, vo_s[...], preferred_element_type=F32)
        s_pos = (fo_s[r, :] + got_s[...]) > 0.0
        m = jnp.where(s_pos, adjh, jnp.zeros_like(adjh))
        nm = jnp.dot(m, uvo_s[...], preferred_element_type=F32)
        acc = (efo_s[r, :] * nm[:, :w]
               + efao_s[r, :] * (av - nm[:, w:]))
        xo = _elu(acc[:, :nclass] / acc[:, nclass:])
        mx = jnp.max(xo, axis=1, keepdims=True)
        lse = mx + jnp.log(jnp.sum(jnp.exp(xo - mx), axis=1, keepdims=True))
        xo_ref[0] = xo - lse


def kernel(x, adj, W_heads, a_heads, W_out, a_out, gc_weight, gc_bias, enc_W, enc_b):
    n, nfeat = x.shape
    nheads, _, nhid = W_heads.shape
    nh = nheads * nhid
    nclass = W_out.shape[1]
    nstruc = enc_W.shape[1]
    wo = nclass + 1

    wcat = jnp.transpose(W_heads, (1, 0, 2)).reshape(nfeat, nh)
    a1 = a_heads[:, 0, :nhid]
    a2 = a_heads[:, 0, nhid:]
    eye = jnp.eye(nheads, dtype=F32)
    A1 = (a1[:, :, None] * eye[:, None, :]).reshape(nh, nheads)
    A2 = (a2[:, :, None] * eye[:, None, :]).reshape(nh, nheads)
    ao1 = a_out[0, :nclass].reshape(nclass, 1)
    ao2 = a_out[0, nclass:].reshape(nclass, 1)
    gcb = gc_bias.reshape(1, nh)
    encb = enc_b.reshape(1, nstruc)
    R8 = jnp.repeat(eye, nhid, axis=1)

    nblk = n // BLK
    fixed = lambda rr, cc: pl.BlockSpec((rr, cc), lambda p, i: (0, 0))

    y3, z3, xo3 = pl.pallas_call(
        _gat_body,
        grid=(3, nblk),
        in_specs=[pl.BlockSpec((BLK, n), lambda p, i: (i * (p == 0), 0)),
                  fixed(n, nfeat),
                  fixed(nfeat, nh),
                  fixed(nh, nheads),
                  fixed(nh, nheads),
                  fixed(nheads, nh),
                  fixed(nh, nh),
                  fixed(1, nh),
                  fixed(nh, nclass),
                  fixed(nh, nstruc),
                  fixed(1, nstruc),
                  fixed(nclass, 1),
                  fixed(nclass, 1)],
        out_specs=[pl.BlockSpec((1, BLK, nstruc), lambda p, i: (p, i, 0)),
                   pl.BlockSpec((1, BLK, nstruc), lambda p, i: (p, i, 0)),
                   pl.BlockSpec((1, BLK, nclass), lambda p, i: (p, i, 0))],
        out_shape=[jax.ShapeDtypeStruct((3, n, nstruc), F32),
                   jax.ShapeDtypeStruct((3, n, nstruc), F32),
                   jax.ShapeDtypeStruct((3, n, nclass), F32)],
        scratch_shapes=[pltpu.VMEM((n, n), BF16),
                        pltpu.VMEM((n, nheads), F32),
                        pltpu.VMEM((n, nheads), F32),
                        pltpu.VMEM((n, nheads), F32),
                        pltpu.VMEM((nheads, n), F32),
                        pltpu.VMEM((n, nh + nheads), BF16),
                        pltpu.VMEM((n, nh + nheads), BF16),
                        pltpu.VMEM((n, nh), F32),
                        pltpu.VMEM((n, nh), BF16),
                        pltpu.VMEM((n, nh), F32),
                        pltpu.VMEM((n, 1), F32),
                        pltpu.VMEM((n, 1), F32),
                        pltpu.VMEM((n, 1), F32),
                        pltpu.VMEM((n, 1), F32),
                        pltpu.VMEM((1, n), F32),
                        pltpu.VMEM((n, 2 * wo), BF16),
                        pltpu.VMEM((n, wo), BF16)],
    )(adj, x, wcat, A1, A2, R8, gc_weight, gcb, W_out, enc_W, encb, ao1, ao2)

    return (xo3[2], y3[0], z3[1])


# speedup vs baseline: 4863.9724x; 1.0774x over previous
import jax
import jax.numpy as jnp
from jax.experimental import pallas as pl
from jax.experimental.pallas import tpu as pltpu

ALPHA = 0.2
BLK = 512
BF16 = jnp.bfloat16
F32 = jnp.float32


def _elu(v):
    return jnp.where(v > 0.0, v, jnp.exp(v) - 1.0)


def _gat_body(adj_ref, x_ref, wcat_ref, a1_ref, a2_ref, r8_ref,
              gcw_ref, gcb_ref, wout_ref, encw_ref, encb_ref, ao1_ref, ao2_ref,
              y_ref, z_ref, xo_ref,
              adj16_s, f_s, ef_s, efa_s, gt_s, u_s, v_s, h1_s, hg_s, h2_s,
              fo_s, efo_s, efao_s, go_s, got_s, uvo_s, vo_s):
    p = pl.program_id(0)
    i = pl.program_id(1)
    nheads = f_s.shape[1]
    nh = u_s.shape[1] - nheads
    nhid = nh // nheads
    nclass = wout_ref.shape[1]
    w = nclass + 1
    r = pl.ds(i * BLK, BLK)

    @pl.when((p == 0) & (i == 0))
    def _prologue_a():
        h = jnp.dot(x_ref[...], wcat_ref[...], preferred_element_type=F32)
        f = jnp.dot(h, a1_ref[...], preferred_element_type=F32)
        g = jnp.dot(h, a2_ref[...], preferred_element_type=F32)
        f_s[...] = f.astype(BF16)
        ef_s[...] = jnp.exp(-f)
        efa_s[...] = jnp.exp(-ALPHA * f)
        gt_s[...] = jnp.transpose(g.astype(BF16))
        eg = jnp.exp(-g)
        ega = jnp.exp(-ALPHA * g)
        u_s[:, :nh] = (jnp.dot(eg, r8_ref[...],
                               preferred_element_type=F32) * h).astype(BF16)
        u_s[:, nh:] = eg.astype(BF16)
        v_s[:, :nh] = (jnp.dot(ega, r8_ref[...],
                               preferred_element_type=F32) * h).astype(BF16)
        v_s[:, nh:] = ega.astype(BF16)

    @pl.when(p == 0)
    def _phase0():
        adjh = adj_ref[...].astype(BF16)
        adj16_s[r, :] = adjh
        u16 = u_s[...]
        v16 = v_s[...]
        fr = f_s[r, :]
        efr = ef_s[r, :]
        efar = efa_s[r, :]
        av = jnp.dot(adjh, v16, preferred_element_type=F32)
        for t in range(nheads):
            s_pos = (fr[:, t:t + 1] + gt_s[t:t + 1, :]) > 0.0
            m = jnp.where(s_pos, adjh, jnp.zeros_like(adjh))
            nu = jnp.dot(m, u16, preferred_element_type=F32)
            nv = jnp.dot(m, v16, preferred_element_type=F32)
            ef = efr[:, t:t + 1]
            efa = efar[:, t:t + 1]
            lo, hi = t * nhid, (t + 1) * nhid
            num = ef * nu[:, lo:hi] + efa * (av[:, lo:hi] - nv[:, lo:hi])
            den = (ef * nu[:, nh + t:nh + t + 1]
                   + efa * (av[:, nh + t:nh + t + 1] - nv[:, nh + t:nh + t + 1]))
            h1_s[r, lo:hi] = _elu(num / den)
        y_ref[0] = jnp.dot(h1_s[r, :], encw_ref[...],
                           preferred_element_type=F32) + encb_ref[...]

    @pl.when((p == 1) & (i == 0))
    def _prologue_b():
        hg_s[...] = jnp.dot(h1_s[...], gcw_ref[...],
                            preferred_element_type=F32).astype(BF16)

    @pl.when(p == 1)
    def _phase1():
        h2 = jnp.dot(adj16_s[r, :], hg_s[...],
                     preferred_element_type=F32) + gcb_ref[...]
        h2_s[r, :] = h2
        z_ref[0] = jnp.dot(h2, encw_ref[...],
                           preferred_element_type=F32) + encb_ref[...]
        ho = jnp.dot(h2, wout_ref[...], preferred_element_type=F32)
        fo = jnp.dot(ho, ao1_ref[...], preferred_element_type=F32)
        go = jnp.dot(ho, ao2_ref[...], preferred_element_type=F32)
        fo_s[r, :] = fo.astype(BF16)
        go_s[r, :] = go.astype(BF16)
        efo_s[r, :] = jnp.exp(-fo)
        efao_s[r, :] = jnp.exp(-ALPHA * fo)
        ego = jnp.exp(-go)
        egao = jnp.exp(-ALPHA * go)
        uvo_s[r, :nclass] = (ego * ho).astype(BF16)
        uvo_s[r, nclass:w] = ego.astype(BF16)
        uvo_s[r, w:w + nclass] = (egao * ho).astype(BF16)
        uvo_s[r, w + nclass:] = egao.astype(BF16)
        vo_s[r, :nclass] = (egao * ho).astype(BF16)
        vo_s[r, nclass:] = egao.astype(BF16)

    @pl.when((p == 2) & (i == 0))
    def _transpose_go():
        got_s[...] = jnp.transpose(go_s[...])

    @pl.when(p == 2)
    def _phase2():
        adjh = adj16_s[r, :]
        av = jnp.dot(adjh, vo_s[...], preferred_element_type=F32)
        s_pos = (fo_s[r, :] + got_s[...]) > 0.0
        m = jnp.where(s_pos, adjh, jnp.zeros_like(adjh))
        nm = jnp.dot(m, uvo_s[...], preferred_element_type=F32)
        acc = (efo_s[r, :] * nm[:, :w]
               + efao_s[r, :] * (av - nm[:, w:]))
        xo = _elu(acc[:, :nclass] / acc[:, nclass:])
        mx = jnp.max(xo, axis=1, keepdims=True)
        lse = mx + jnp.log(jnp.sum(jnp.exp(xo - mx), axis=1, keepdims=True))
        xo_ref[0] = xo - lse


def kernel(x, adj, W_heads, a_heads, W_out, a_out, gc_weight, gc_bias, enc_W, enc_b):
    n, nfeat = x.shape
    nheads, _, nhid = W_heads.shape
    nh = nheads * nhid
    nclass = W_out.shape[1]
    nstruc = enc_W.shape[1]
    wo = nclass + 1

    wcat = jnp.transpose(W_heads, (1, 0, 2)).reshape(nfeat, nh)
    a1 = a_heads[:, 0, :nhid]
    a2 = a_heads[:, 0, nhid:]
    eye = jnp.eye(nheads, dtype=F32)
    A1 = (a1[:, :, None] * eye[:, None, :]).reshape(nh, nheads)
    A2 = (a2[:, :, None] * eye[:, None, :]).reshape(nh, nheads)
    ao1 = a_out[0, :nclass].reshape(nclass, 1)
    ao2 = a_out[0, nclass:].reshape(nclass, 1)
    gcb = gc_bias.reshape(1, nh)
    encb = enc_b.reshape(1, nstruc)
    R8 = jnp.repeat(eye, nhid, axis=1)

    nblk = n // BLK
    fixed = lambda rr, cc: pl.BlockSpec((rr, cc), lambda p, i: (0, 0))

    y3, z3, xo3 = pl.pallas_call(
        _gat_body,
        grid=(3, nblk),
        in_specs=[pl.BlockSpec((BLK, n), lambda p, i: (i * (p == 0), 0)),
                  fixed(n, nfeat),
                  fixed(nfeat, nh),
                  fixed(nh, nheads),
                  fixed(nh, nheads),
                  fixed(nheads, nh),
                  fixed(nh, nh),
                  fixed(1, nh),
                  fixed(nh, nclass),
                  fixed(nh, nstruc),
                  fixed(1, nstruc),
                  fixed(nclass, 1),
                  fixed(nclass, 1)],
        out_specs=[pl.BlockSpec((1, BLK, nstruc), lambda p, i: (p, i, 0)),
                   pl.BlockSpec((1, BLK, nstruc), lambda p, i: (p, i, 0)),
                   pl.BlockSpec((1, BLK, nclass), lambda p, i: (p, i, 0))],
        out_shape=[jax.ShapeDtypeStruct((3, n, nstruc), F32),
                   jax.ShapeDtypeStruct((3, n, nstruc), F32),
                   jax.ShapeDtypeStruct((3, n, nclass), F32)],
        scratch_shapes=[pltpu.VMEM((n, n), BF16),
                        pltpu.VMEM((n, nheads), BF16),
                        pltpu.VMEM((n, nheads), F32),
                        pltpu.VMEM((n, nheads), F32),
                        pltpu.VMEM((nheads, n), BF16),
                        pltpu.VMEM((n, nh + nheads), BF16),
                        pltpu.VMEM((n, nh + nheads), BF16),
                        pltpu.VMEM((n, nh), F32),
                        pltpu.VMEM((n, nh), BF16),
                        pltpu.VMEM((n, nh), F32),
                        pltpu.VMEM((n, 1), BF16),
                        pltpu.VMEM((n, 1), F32),
                        pltpu.VMEM((n, 1), F32),
                        pltpu.VMEM((n, 1), BF16),
                        pltpu.VMEM((1, n), BF16),
                        pltpu.VMEM((n, 2 * wo), BF16),
                        pltpu.VMEM((n, wo), BF16)],
    )(adj, x, wcat, A1, A2, R8, gc_weight, gcb, W_out, enc_W, encb, ao1, ao2)

    return (xo3[2], y3[0], z3[1])


# speedup vs baseline: 6981.3208x; 1.4353x over previous
import jax
import jax.numpy as jnp
from jax.experimental import pallas as pl
from jax.experimental.pallas import tpu as pltpu

ALPHA = 0.2
BLK = 512
BF16 = jnp.bfloat16
F32 = jnp.float32


def _elu(v):
    return jnp.where(v > 0.0, v, jnp.exp(v) - 1.0)


def _gat_body(adj_ref, x_ref, wcat_ref, a1_ref, a2_ref, r8_ref,
              gcw_ref, gcb_ref, wout_ref, encw_ref, encb_ref, ao1_ref, ao2_ref,
              y_ref, z_ref, xo_ref,
              adj16_s, eb_s, cbt_s, v_s, h1_s, hg_s, h2_s,
              ebo_s, cbo_s, cbot_s, vo_s):
    p = pl.program_id(0)
    i = pl.program_id(1)
    nheads = eb_s.shape[1]
    nh = v_s.shape[1] - nheads
    nhid = nh // nheads
    nclass = wout_ref.shape[1]
    w = nclass + 1
    beta = 1.0 - ALPHA
    r = pl.ds(i * BLK, BLK)

    @pl.when((p == 0) & (i == 0))
    def _prologue_a():
        h = jnp.dot(x_ref[...], wcat_ref[...], preferred_element_type=F32)
        f = jnp.dot(h, a1_ref[...], preferred_element_type=F32)
        g = jnp.dot(h, a2_ref[...], preferred_element_type=F32)
        eb_s[...] = jnp.exp(-beta * f).astype(BF16)
        cbt_s[...] = jnp.transpose(jnp.exp(-beta * g).astype(BF16))
        ega = jnp.exp(-ALPHA * g)
        v_s[:, :nh] = (jnp.dot(ega, r8_ref[...],
                               preferred_element_type=F32) * h).astype(BF16)
        v_s[:, nh:] = ega.astype(BF16)

    @pl.when(p == 0)
    def _phase0():
        adjh = adj_ref[...].astype(BF16)
        adj16_s[r, :] = adjh
        v16 = v_s[...]
        ebr = eb_s[r, :]
        one = jnp.ones((), BF16)
        for t in range(nheads):
            q = ebr[:, t:t + 1] * cbt_s[t:t + 1, :]
            e1 = jnp.minimum(q, one) * adjh
            nv = jnp.dot(e1, v16, preferred_element_type=F32)
            lo, hi = t * nhid, (t + 1) * nhid
            h1_s[r, lo:hi] = _elu(nv[:, lo:hi] / nv[:, nh + t:nh + t + 1])
        y_ref[0] = jnp.dot(h1_s[r, :], encw_ref[...],
                           preferred_element_type=F32) + encb_ref[...]

    @pl.when((p == 1) & (i == 0))
    def _prologue_b():
        hg_s[...] = jnp.dot(h1_s[...], gcw_ref[...],
                            preferred_element_type=F32).astype(BF16)

    @pl.when(p == 1)
    def _phase1():
        h2 = jnp.dot(adj16_s[r, :], hg_s[...],
                     preferred_element_type=F32) + gcb_ref[...]
        h2_s[r, :] = h2
        z_ref[0] = jnp.dot(h2, encw_ref[...],
                           preferred_element_type=F32) + encb_ref[...]
        ho = jnp.dot(h2, wout_ref[...], preferred_element_type=F32)
        fo = jnp.dot(ho, ao1_ref[...], preferred_element_type=F32)
        go = jnp.dot(ho, ao2_ref[...], preferred_element_type=F32)
        ebo_s[r, :] = jnp.exp(-beta * fo).astype(BF16)
        cbo_s[r, :] = jnp.exp(-beta * go).astype(BF16)
        egao = jnp.exp(-ALPHA * go)
        vo_s[r, :nclass] = (egao * ho).astype(BF16)
        vo_s[r, nclass:] = egao.astype(BF16)

    @pl.when((p == 2) & (i == 0))
    def _transpose_go():
        cbot_s[...] = jnp.transpose(cbo_s[...])

    @pl.when(p == 2)
    def _phase2():
        adjh = adj16_s[r, :]
        q = ebo_s[r, :] * cbot_s[...]
        e1 = jnp.minimum(q, jnp.ones((), BF16)) * adjh
        nv = jnp.dot(e1, vo_s[...], preferred_element_type=F32)
        xo = _elu(nv[:, :nclass] / nv[:, nclass:])
        mx = jnp.max(xo, axis=1, keepdims=True)
        lse = mx + jnp.log(jnp.sum(jnp.exp(xo - mx), axis=1, keepdims=True))
        xo_ref[0] = xo - lse


def kernel(x, adj, W_heads, a_heads, W_out, a_out, gc_weight, gc_bias, enc_W, enc_b):
    n, nfeat = x.shape
    nheads, _, nhid = W_heads.shape
    nh = nheads * nhid
    nclass = W_out.shape[1]
    nstruc = enc_W.shape[1]
    wo = nclass + 1

    wcat = jnp.transpose(W_heads, (1, 0, 2)).reshape(nfeat, nh)
    a1 = a_heads[:, 0, :nhid]
    a2 = a_heads[:, 0, nhid:]
    eye = jnp.eye(nheads, dtype=F32)
    A1 = (a1[:, :, None] * eye[:, None, :]).reshape(nh, nheads)
    A2 = (a2[:, :, None] * eye[:, None, :]).reshape(nh, nheads)
    ao1 = a_out[0, :nclass].reshape(nclass, 1)
    ao2 = a_out[0, nclass:].reshape(nclass, 1)
    gcb = gc_bias.reshape(1, nh)
    encb = enc_b.reshape(1, nstruc)
    R8 = jnp.repeat(eye, nhid, axis=1)

    nblk = n // BLK
    fixed = lambda rr, cc: pl.BlockSpec((rr, cc), lambda p, i: (0, 0))

    y3, z3, xo3 = pl.pallas_call(
        _gat_body,
        grid=(3, nblk),
        in_specs=[pl.BlockSpec((BLK, n), lambda p, i: (i * (p == 0), 0)),
                  fixed(n, nfeat),
                  fixed(nfeat, nh),
                  fixed(nh, nheads),
                  fixed(nh, nheads),
                  fixed(nheads, nh),
                  fixed(nh, nh),
                  fixed(1, nh),
                  fixed(nh, nclass),
                  fixed(nh, nstruc),
                  fixed(1, nstruc),
                  fixed(nclass, 1),
                  fixed(nclass, 1)],
        out_specs=[pl.BlockSpec((1, BLK, nstruc), lambda p, i: (p, i, 0)),
                   pl.BlockSpec((1, BLK, nstruc), lambda p, i: (p, i, 0)),
                   pl.BlockSpec((1, BLK, nclass), lambda p, i: (p, i, 0))],
        out_shape=[jax.ShapeDtypeStruct((3, n, nstruc), F32),
                   jax.ShapeDtypeStruct((3, n, nstruc), F32),
                   jax.ShapeDtypeStruct((3, n, nclass), F32)],
        scratch_shapes=[pltpu.VMEM((n, n), BF16),
                        pltpu.VMEM((n, nheads), BF16),
                        pltpu.VMEM((nheads, n), BF16),
                        pltpu.VMEM((n, nh + nheads), BF16),
                        pltpu.VMEM((n, nh), F32),
                        pltpu.VMEM((n, nh), BF16),
                        pltpu.VMEM((n, nh), F32),
                        pltpu.VMEM((n, 1), BF16),
                        pltpu.VMEM((n, 1), BF16),
                        pltpu.VMEM((1, n), BF16),
                        pltpu.VMEM((n, wo), BF16)],
    )(adj, x, wcat, A1, A2, R8, gc_weight, gcb, W_out, enc_W, encb, ao1, ao2)

    return (xo3[2], y3[0], z3[1])
